# baseline (speedup 1.0000x reference)
import functools

import jax
import jax.numpy as jnp
from jax import lax
from jax.experimental import pallas as pl
from jax.experimental.pallas import tpu as pltpu
from jax.experimental.pallas import tpu_sc as plsc

B = 16384
NUM_ITEMS = 1000000
_PADDED = 1000448

_info = plsc.get_sparse_core_info()
_NC, _NS = _info.num_cores, _info.num_subcores
_NW = _NC * _NS
_CHUNK = 128
_PER_W = B // _NW
_NCH = _PER_W // _CHUNK


@functools.partial(
    pl.kernel,
    mesh=plsc.VectorSubcoreMesh(core_axis_name="c", subcore_axis_name="s"),
    out_type=jax.ShapeDtypeStruct((B,), jnp.float32),
    scratch_types=[
        pltpu.VMEM((_NCH, _CHUNK), jnp.int32),
        pltpu.VMEM((_PER_W,), jnp.float32),
        pltpu.SemaphoreType.DMA,
        pltpu.SemaphoreType.DMA,
    ],
    compiler_params=pltpu.CompilerParams(
        skip_device_barrier=True, disable_bounds_checks=True
    ),
)
def _bias_gather(table_hbm, idx_hbm, out_hbm, idx_v, rows_v, isem, gsem):
    wid = lax.axis_index("s") * _NC + lax.axis_index("c")
    idx_copies = [
        pltpu.async_copy(idx_hbm.at[wid, j], idx_v.at[j], isem)
        for j in range(_NCH)
    ]
    gathers = []
    for j in range(_NCH):
        idx_copies[j].wait()
        gathers.append(
            pltpu.async_copy(
                table_hbm.at[idx_v.at[j]],
                rows_v.at[pl.ds(j * _CHUNK, _CHUNK)],
                gsem,
            )
        )
    for g in gathers:
        g.wait()
    pltpu.sync_copy(rows_v, out_hbm.at[pl.ds(wid * _PER_W, _PER_W)])


def kernel(item_sequences, item_ids, item_biases):
    idx = item_ids.reshape(_NW, _NCH, _CHUNK)
    head = lax.slice(item_biases, (0, 0), (999424, 1)).reshape(999424)
    tail = lax.slice(item_biases, (999424, 0), (NUM_ITEMS, 1)).reshape(576)
    zeros = jnp.zeros((_PADDED - NUM_ITEMS,), jnp.float32)
    table = jnp.concatenate([head, tail, zeros], axis=0)
    out = _bias_gather(table, idx)
    return out.reshape(B, 1)


# speedup vs baseline: 1.8529x; 1.8529x over previous
import functools

import jax
import jax.numpy as jnp
from jax import lax
from jax.experimental import pallas as pl
from jax.experimental.pallas import tpu as pltpu
from jax.experimental.pallas import tpu_sc as plsc

B = 16384
NUM_ITEMS = 1000000
_PADDED = 1000448

_info = plsc.get_sparse_core_info()
_NC, _NS = _info.num_cores, _info.num_subcores
_NW = _NC * _NS
_CHUNK = 128
_PER_W = B // _NW
_NCH = _PER_W // _CHUNK


@functools.partial(
    pl.kernel,
    mesh=plsc.VectorSubcoreMesh(core_axis_name="c", subcore_axis_name="s"),
    out_type=jax.ShapeDtypeStruct((B,), jnp.float32),
    scratch_types=[
        pltpu.VMEM((_NCH, _CHUNK), jnp.int32),
        pltpu.VMEM((_PER_W,), jnp.float32),
        pltpu.SemaphoreType.DMA,
        pltpu.SemaphoreType.DMA,
    ],
    compiler_params=pltpu.CompilerParams(
        skip_device_barrier=True, disable_bounds_checks=True
    ),
)
def _bias_gather(table_hbm, idx_hbm, out_hbm, idx_v, rows_v, isem, gsem):
    wid = lax.axis_index("s") * _NC + lax.axis_index("c")
    idx_copies = [
        pltpu.async_copy(idx_hbm.at[wid, j], idx_v.at[j], isem)
        for j in range(_NCH)
    ]
    gathers = []
    for j in range(_NCH):
        idx_copies[j].wait()
        gathers.append(
            pltpu.async_copy(
                table_hbm.at[idx_v.at[j]],
                rows_v.at[pl.ds(j * _CHUNK, _CHUNK)],
                gsem,
            )
        )
    for g in gathers:
        g.wait()
    pltpu.sync_copy(rows_v, out_hbm.at[pl.ds(wid * _PER_W, _PER_W)])


def kernel(item_sequences, item_ids, item_biases):
    idx = item_ids.reshape(_NW, _NCH, _CHUNK)
    head = lax.slice(item_biases, (0, 0), (999424, 1)).reshape(999424)
    tail = lax.slice(item_biases, (999424, 0), (NUM_ITEMS, 1)).reshape(576)
    tail = jnp.pad(tail, (0, _PADDED - NUM_ITEMS))
    table = jnp.concatenate([head, tail], axis=0)
    out = _bias_gather(table, idx)
    return out.reshape(B, 1)
